# baseline (speedup 1.0000x reference)
import math

import jax
import jax.numpy as jnp
from jax.experimental import pallas as pl
from jax.experimental.pallas import tpu as pltpu

_VOCAB = 32000
_N = 4096
_FILL = 0.1 / (_VOCAB - 2)
_CONF = 1.0 - 0.1
_C_ROW = 0.1 * math.log(_FILL) + _CONF * math.log(_CONF)
_KMUL = _CONF / _FILL

_RB = 1024
_BV = 6400
_GR = _N // _RB
_GV = _VOCAB // _BV


def _body(t_ref, x_ref, out_ref, acc_ref):
    i = pl.program_id(0)
    j = pl.program_id(1)

    @pl.when((i == 0) & (j == 0))
    def _init():
        acc_ref[0] = 0.0
        acc_ref[1] = 0.0
        acc_ref[2] = 0.0
        acc_ref[3] = 0.0

    x = x_ref[...]
    tcol = t_ref[:, 0:1]
    valid = tcol != 0

    lane = jax.lax.broadcasted_iota(jnp.int32, (_RB, _BV), 1)
    rel = tcol - j * _BV
    y = jnp.where(lane == rel, x * _KMUL, x)
    srows = jnp.sum(y, axis=1, keepdims=True)
    acc_ref[0] += jnp.sum(jnp.where(valid, srows, 0.0))

    @pl.when(j == 0)
    def _col0():
        acc_ref[2] += jnp.sum(jnp.where(valid, x[:, 0:1], 0.0))
        acc_ref[3] += jnp.sum(jnp.where(valid, 1.0, 0.0))

    @pl.when((i == _GR - 1) & (j == _GV - 1))
    def _fin():
        out_ref[0, 0] = (acc_ref[3] * _C_ROW - _FILL * acc_ref[0]
                         + _FILL * acc_ref[2])


def kernel(predictions, targets):
    n = predictions.shape[0]
    t2 = jnp.broadcast_to(targets[:, None].astype(jnp.int32), (n, 128))
    out = pl.pallas_call(
        _body,
        grid=(_GR, _GV),
        in_specs=[
            pl.BlockSpec((_RB, 128), lambda i, j: (i, 0)),
            pl.BlockSpec((_RB, _BV), lambda i, j: (i, j)),
        ],
        out_specs=pl.BlockSpec((1, 1), lambda i, j: (0, 0),
                               memory_space=pltpu.SMEM),
        out_shape=jax.ShapeDtypeStruct((1, 1), jnp.float32),
        scratch_shapes=[pltpu.SMEM((4,), jnp.float32)],
        compiler_params=pltpu.CompilerParams(
            dimension_semantics=("arbitrary", "arbitrary")),
    )(t2, predictions)
    return out[0, 0]


# speedup vs baseline: 1.0130x; 1.0130x over previous
import math

import jax
import jax.numpy as jnp
from jax.experimental import pallas as pl
from jax.experimental.pallas import tpu as pltpu

_VOCAB = 32000
_N = 4096
_FILL = 0.1 / (_VOCAB - 2)
_CONF = 1.0 - 0.1
_C_ROW = 0.1 * math.log(_FILL) + _CONF * math.log(_CONF)
_KMUL = _CONF / _FILL

_RB = 256
_BV = 16000
_GR = _N // _RB
_GV = _VOCAB // _BV


def _body(t_ref, x_ref, out_ref, acc_ref):
    i = pl.program_id(0)
    j = pl.program_id(1)

    @pl.when((i == 0) & (j == 0))
    def _init():
        acc_ref[0] = 0.0
        acc_ref[1] = 0.0
        acc_ref[2] = 0.0
        acc_ref[3] = 0.0

    x = x_ref[...]
    tcol = t_ref[:, 0:1]
    valid = tcol != 0

    lane = jax.lax.broadcasted_iota(jnp.int32, (_RB, _BV), 1)
    rel = tcol - j * _BV
    y = jnp.where(lane == rel, x * _KMUL, x)
    srows = jnp.sum(y, axis=1, keepdims=True)
    acc_ref[0] += jnp.sum(jnp.where(valid, srows, 0.0))

    @pl.when(j == 0)
    def _col0():
        acc_ref[2] += jnp.sum(jnp.where(valid, x[:, 0:1], 0.0))
        acc_ref[3] += jnp.sum(jnp.where(valid, 1.0, 0.0))

    @pl.when((i == _GR - 1) & (j == _GV - 1))
    def _fin():
        out_ref[0, 0] = (acc_ref[3] * _C_ROW - _FILL * acc_ref[0]
                         + _FILL * acc_ref[2])


def kernel(predictions, targets):
    n = predictions.shape[0]
    t2 = jnp.broadcast_to(targets[:, None].astype(jnp.int32), (n, 128))
    out = pl.pallas_call(
        _body,
        grid=(_GR, _GV),
        in_specs=[
            pl.BlockSpec((_RB, 128), lambda i, j: (i, 0)),
            pl.BlockSpec((_RB, _BV), lambda i, j: (i, j)),
        ],
        out_specs=pl.BlockSpec((1, 1), lambda i, j: (0, 0),
                               memory_space=pltpu.SMEM),
        out_shape=jax.ShapeDtypeStruct((1, 1), jnp.float32),
        scratch_shapes=[pltpu.SMEM((4,), jnp.float32)],
        compiler_params=pltpu.CompilerParams(
            dimension_semantics=("arbitrary", "arbitrary")),
    )(t2, predictions)
    return out[0, 0]


# speedup vs baseline: 1.0138x; 1.0007x over previous
import math

import jax
import jax.numpy as jnp
from jax.experimental import pallas as pl
from jax.experimental.pallas import tpu as pltpu

_VOCAB = 32000
_N = 4096
_FILL = 0.1 / (_VOCAB - 2)
_CONF = 1.0 - 0.1
_C_ROW = 0.1 * math.log(_FILL) + _CONF * math.log(_CONF)
_KMUL = _CONF / _FILL

_RB = 128
_BV = 32000
_GR = _N // _RB
_GV = _VOCAB // _BV


def _body(t_ref, x_ref, out_ref, acc_ref):
    i = pl.program_id(0)
    j = pl.program_id(1)

    @pl.when((i == 0) & (j == 0))
    def _init():
        acc_ref[0] = 0.0
        acc_ref[1] = 0.0
        acc_ref[2] = 0.0
        acc_ref[3] = 0.0

    x = x_ref[...]
    tcol = t_ref[:, 0:1]
    valid = tcol != 0

    lane = jax.lax.broadcasted_iota(jnp.int32, (_RB, _BV), 1)
    rel = tcol - j * _BV
    y = jnp.where(lane == rel, x * _KMUL, x)
    srows = jnp.sum(y, axis=1, keepdims=True)
    acc_ref[0] += jnp.sum(jnp.where(valid, srows, 0.0))

    @pl.when(j == 0)
    def _col0():
        acc_ref[2] += jnp.sum(jnp.where(valid, x[:, 0:1], 0.0))
        acc_ref[3] += jnp.sum(jnp.where(valid, 1.0, 0.0))

    @pl.when((i == _GR - 1) & (j == _GV - 1))
    def _fin():
        out_ref[0, 0] = (acc_ref[3] * _C_ROW - _FILL * acc_ref[0]
                         + _FILL * acc_ref[2])


def kernel(predictions, targets):
    n = predictions.shape[0]
    t2 = jnp.broadcast_to(targets[:, None].astype(jnp.int32), (n, 128))
    out = pl.pallas_call(
        _body,
        grid=(_GR, _GV),
        in_specs=[
            pl.BlockSpec((_RB, 128), lambda i, j: (i, 0)),
            pl.BlockSpec((_RB, _BV), lambda i, j: (i, j)),
        ],
        out_specs=pl.BlockSpec((1, 1), lambda i, j: (0, 0),
                               memory_space=pltpu.SMEM),
        out_shape=jax.ShapeDtypeStruct((1, 1), jnp.float32),
        scratch_shapes=[pltpu.SMEM((4,), jnp.float32)],
        compiler_params=pltpu.CompilerParams(
            dimension_semantics=("arbitrary", "arbitrary")),
    )(t2, predictions)
    return out[0, 0]
